# baseline (speedup 1.0000x reference)
import jax
import jax.numpy as jnp
from jax import lax
from jax.experimental import pallas as pl
from jax.experimental.pallas import tpu as pltpu
from jax.experimental.pallas import tpu_sc as plsc

NUM_CATEGORIES = 1000000
L = 26
D = 16
B = 16384
N = B * L

NC = 2
NS = 16
NW = NC * NS
PER_W = N // NW

IDX_ROW = 128
IDX_ROWS_W = PER_W // IDX_ROW
CHUNK = 1664
DMAS_PER_CHUNK = CHUNK // IDX_ROW
CHUNKS = PER_W // CHUNK


def _body(x_hbm, table_hbm, out_hbm, idx_v, buf_v, gsem):
    wid = lax.axis_index("s") * NC + lax.axis_index("c")
    base_idx_row = wid * IDX_ROWS_W
    base_out = wid * PER_W

    pltpu.sync_copy(x_hbm.at[pl.ds(base_idx_row, IDX_ROWS_W)], idx_v)

    def fire(c, p):
        for j in range(DMAS_PER_CHUNK):
            pltpu.async_copy(
                table_hbm.at[idx_v.at[c * DMAS_PER_CHUNK + j]],
                buf_v.at[p, pl.ds(j * IDX_ROW, IDX_ROW)],
                gsem.at[p])

    def drain(c, p):
        pltpu.make_async_copy(
            out_hbm.at[pl.ds(base_out + c * CHUNK, CHUNK)], buf_v.at[p],
            gsem.at[p]).wait()

    fire(0, 0)

    def chunk_body(c, carry):
        p = lax.rem(c, 2)

        @pl.when(c + 1 < CHUNKS)
        def _():
            fire(c + 1, 1 - p)

        drain(c, p)
        pltpu.sync_copy(buf_v.at[p], out_hbm.at[pl.ds(base_out + c * CHUNK, CHUNK)])
        return carry

    lax.fori_loop(0, CHUNKS, chunk_body, 0)


@jax.jit
def kernel(x, shared_embed, position_weights, position_bias):
    x_flat = x.reshape(N // IDX_ROW, IDX_ROW)
    mesh = plsc.VectorSubcoreMesh(core_axis_name="c", subcore_axis_name="s")
    flat = pl.kernel(
        _body,
        out_type=jax.ShapeDtypeStruct((N, D), jnp.float32),
        mesh=mesh,
        compiler_params=pltpu.CompilerParams(use_tc_tiling_on_sc=False),
        scratch_types=[
            pltpu.VMEM((IDX_ROWS_W, IDX_ROW), jnp.int32),
            pltpu.VMEM((2, CHUNK, D), jnp.float32),
            pltpu.SemaphoreType.DMA((2,)),
        ],
    )(x_flat, shared_embed)
    return flat.reshape(B, L, D) * position_weights + position_bias


# speedup vs baseline: 1.1236x; 1.1236x over previous
import jax
import jax.numpy as jnp
from jax import lax
from jax.experimental import pallas as pl
from jax.experimental.pallas import tpu as pltpu
from jax.experimental.pallas import tpu_sc as plsc

NUM_CATEGORIES = 1000000
L = 26
D = 16
B = 16384
N = B * L

NC = 2
NS = 16
NW = NC * NS
PER_W = N // NW

IDX_ROW = 128
IDX_ROWS_W = PER_W // IDX_ROW
CHUNK = 1664
DMAS_PER_CHUNK = CHUNK // IDX_ROW
CHUNKS = PER_W // CHUNK
OUT_COLS = 128
OUT_ROWS = N * D // OUT_COLS
OUT_ROWS_C = CHUNK * D // OUT_COLS


def _body(x_hbm, table_hbm, out_hbm, idx_v, buf_v, gsem):
    wid = lax.axis_index("s") * NC + lax.axis_index("c")
    base_idx_row = wid * IDX_ROWS_W
    base_out = wid * PER_W

    pltpu.sync_copy(x_hbm.at[pl.ds(base_idx_row, IDX_ROWS_W)], idx_v)

    def fire(c, p):
        for j in range(DMAS_PER_CHUNK):
            pltpu.async_copy(
                table_hbm.at[idx_v.at[c * DMAS_PER_CHUNK + j]],
                buf_v.at[p, pl.ds(j * IDX_ROW, IDX_ROW)],
                gsem.at[p])

    def drain(c, p):
        pltpu.make_async_copy(
            table_hbm.at[pl.ds(0, CHUNK)], buf_v.at[p],
            gsem.at[p]).wait()

    fire(0, 0)

    def chunk_body(c, carry):
        p = lax.rem(c, 2)

        @pl.when(c + 1 < CHUNKS)
        def _():
            fire(c + 1, 1 - p)

        drain(c, p)
        pltpu.sync_copy(buf_v.at[p],
                        out_hbm.at[pl.ds(base_out + c * CHUNK, CHUNK)])
        return carry

    lax.fori_loop(0, CHUNKS, chunk_body, 0)


BB = 512


def _tc_body(flat_ref, pw_ref, pb_ref, out_ref):
    r = flat_ref[...].reshape(BB, L, D)
    out_ref[...] = r * pw_ref[...] + pb_ref[...]


@jax.jit
def kernel(x, shared_embed, position_weights, position_bias):
    x_flat = x.reshape(N // IDX_ROW, IDX_ROW)
    mesh = plsc.VectorSubcoreMesh(core_axis_name="c", subcore_axis_name="s")
    flat = pl.kernel(
        _body,
        out_type=jax.ShapeDtypeStruct((N, D), jnp.float32),
        mesh=mesh,
        compiler_params=pltpu.CompilerParams(use_tc_tiling_on_sc=False),
        scratch_types=[
            pltpu.VMEM((IDX_ROWS_W, IDX_ROW), jnp.int32),
            pltpu.VMEM((2, CHUNK, D), jnp.float32),
            pltpu.SemaphoreType.DMA((2,)),
        ],
    )(x_flat, shared_embed)
    return pl.pallas_call(
        _tc_body,
        out_shape=jax.ShapeDtypeStruct((B, L, D), jnp.float32),
        grid=(B // BB,),
        in_specs=[
            pl.BlockSpec((BB * L, D), lambda i: (i, 0)),
            pl.BlockSpec((L, D), lambda i: (0, 0)),
            pl.BlockSpec((L, D), lambda i: (0, 0)),
        ],
        out_specs=pl.BlockSpec((BB, L, D), lambda i: (i, 0, 0)),
    )(flat, position_weights, position_bias)
